# Initial kernel scaffold; baseline (speedup 1.0000x reference)
import jax
import jax.numpy as jnp
from jax.experimental import pallas as pl


def kernel(data_encoded, data_decoded, data_true, label_true, batch_size):
    raise NotImplementedError("write your pallas kernel here")


# speedup vs baseline: 17.4008x; 17.4008x over previous
import jax
import jax.numpy as jnp
from jax import lax
from jax.experimental import pallas as pl
from jax.experimental.pallas import tpu as pltpu

_BINS = 50
_RATIO_KLD = 0.5
_GROUPS = [(7, 19), (19, 21), (21, 25), (25, 27), (27, 29), (29, 31),
           (31, 34), (34, 38), (38, 50)]


def _loss_kernel(de_ref, dd_ref, dt_ref, lt_ref, out_ref):
    B = de_ref.shape[0]
    dd = dd_ref[...]
    dt = dt_ref[...]
    de = de_ref[...]
    marital = lt_ref[...][:, 1:2]

    diff = dd[:, 0:7] - dt[:, 0:7]
    numerical_loss = jnp.sum(diff * diff) / (B * 7)
    mse_loss = numerical_loss * 7.0

    ce_loss = jnp.float32(0.0)
    for (s, e) in _GROUPS:
        z = dd[:, s:e]
        t = dt[:, s:e]
        m = jnp.max(z, axis=1, keepdims=True)
        sh = z - m
        lse = jnp.log(jnp.sum(jnp.exp(sh), axis=1))
        picked = jnp.sum(t * sh, axis=1) - lse
        ce_loss = ce_loss + (-jnp.mean(picked))

    wS = 1.0 - marital
    wM = marital
    n_s = jnp.sum(wS)
    n_m = jnp.sum(wM)

    lo = jnp.min(de, axis=0)
    hi = jnp.max(de, axis=0)
    flat = hi == lo
    lo = jnp.where(flat, lo - 0.5, lo)
    hi = jnp.where(flat, hi + 0.5, hi)

    step = lax.broadcasted_iota(jnp.int32, (1, _BINS), 1).astype(jnp.float32) / float(_BINS)
    edges = lo[:, None] * (1.0 - step) + hi[:, None] * step

    s_rows = []
    m_rows = []
    for i in range(10):
        x = de[:, i:i + 1]
        e_in = edges[i:i + 1, 1:_BINS]
        cmp = (x >= e_in).astype(jnp.float32)
        Ss = jnp.sum(cmp * wS, axis=0)
        Sm = jnp.sum(cmp * wM, axis=0)
        z1 = jnp.zeros((1,), jnp.float32)
        cs = jnp.concatenate([n_s[None], Ss]) - jnp.concatenate([Ss, z1])
        cm = jnp.concatenate([n_m[None], Sm]) - jnp.concatenate([Sm, z1])
        s_rows.append(cs / n_s)
        m_rows.append(cm / n_m)
    p = jnp.stack(s_rows)
    q = jnp.stack(m_rows)
    kld = jnp.sum(jnp.where(p > 0, p * jnp.log(p / (q + 1e-10)), 0.0))

    alpha = jnp.float32(_RATIO_KLD)
    multi = (1.0 - alpha) * (mse_loss + ce_loss) + alpha * kld
    out_ref[0] = multi
    out_ref[1] = mse_loss
    out_ref[2] = ce_loss
    out_ref[3] = alpha * kld


def kernel(data_encoded, data_decoded, data_true, label_true, batch_size):
    del batch_size
    out = pl.pallas_call(
        _loss_kernel,
        out_shape=jax.ShapeDtypeStruct((4,), jnp.float32),
        in_specs=[
            pl.BlockSpec(memory_space=pltpu.VMEM),
            pl.BlockSpec(memory_space=pltpu.VMEM),
            pl.BlockSpec(memory_space=pltpu.VMEM),
            pl.BlockSpec(memory_space=pltpu.VMEM),
        ],
        out_specs=pl.BlockSpec(memory_space=pltpu.SMEM),
    )(data_encoded, data_decoded, data_true, label_true)
    return (out[0], out[1], out[2], out[3])
